# baseline (speedup 1.0000x reference)
import functools
import jax
import jax.numpy as jnp
from jax.experimental import pallas as pl
from jax.experimental.pallas import tpu as pltpu

N = 2048
D = 768
H = 3072
O = 768
E = 8
K = 2

BN = 256
NEG = -1e30


def _gate_body(x_ref, g1_ref, gb1_ref, g2_ref, gb2_ref, g3_ref, gb3_ref,
               p_ref):
    x = x_ref[...]
    h1 = jnp.maximum(
        jnp.dot(x, g1_ref[...], preferred_element_type=jnp.float32)
        + gb1_ref[...], 0.0)
    h2 = jnp.maximum(
        jnp.dot(h1, g2_ref[...], preferred_element_type=jnp.float32)
        + gb2_ref[...], 0.0)
    logits = jnp.dot(h2, g3_ref[...], preferred_element_type=jnp.float32) \
        + gb3_ref[...]
    lane = jax.lax.broadcasted_iota(jnp.int32, logits.shape, 1)
    lm = jnp.where(lane < E, logits, NEG)
    m1 = jnp.max(lm, axis=1, keepdims=True)
    i1 = jnp.min(jnp.where(lm == m1, lane, 10**6), axis=1, keepdims=True)
    lm2 = jnp.where(lane == i1, NEG, lm)
    m2 = jnp.max(lm2, axis=1, keepdims=True)
    i2 = jnp.min(jnp.where(lm2 == m2, lane, 10**6), axis=1, keepdims=True)
    ed = jnp.exp(m2 - m1)
    denom = 1.0 + ed
    p1 = 1.0 / denom
    p2 = ed / denom
    p = jnp.where(lane == i1, p1, jnp.where(lane == i2, p2, 0.0))
    p_ref[...] = p[:, :E]


def _expert_body(p_ref, x_ref, w1_ref, b1_ref, w2_ref, b2_ref, y_ref,
                 acc_ref):
    e = pl.program_id(0)
    i = pl.program_id(1)
    xb = x_ref[...]
    he = jnp.dot(xb, w1_ref[0], preferred_element_type=jnp.float32)
    he = jnp.maximum(he + b1_ref[0], 0.0).astype(jnp.bfloat16)
    out = jnp.dot(he, w2_ref[0], preferred_element_type=jnp.float32)
    out = out + b2_ref[0]
    oh = (jax.lax.broadcasted_iota(jnp.int32, (E, O), 0) == e).astype(jnp.float32)
    pe = jnp.dot(p_ref[...], oh, preferred_element_type=jnp.float32)
    rows = pl.ds(i * BN, BN)

    @pl.when(e == 0)
    def _():
        acc_ref[rows, :] = pe * out

    @pl.when(e > 0)
    def _():
        acc_ref[rows, :] = acc_ref[rows, :] + pe * out

    y_ref[...] = acc_ref[rows, :]


@jax.jit
def kernel(x, W1, b1, W2, b2, g1, gb1, g2, gb2, g3, gb3):
    g3p = jnp.zeros((D, 128), jnp.float32).at[:, :E].set(g3)
    gb3p = jnp.zeros((1, 128), jnp.float32).at[0, :E].set(gb3)

    p = pl.pallas_call(
        _gate_body,
        grid=(N // BN,),
        in_specs=[
            pl.BlockSpec((BN, D), lambda i: (i, 0)),
            pl.BlockSpec((D, 4 * D), lambda i: (0, 0)),
            pl.BlockSpec((1, 4 * D), lambda i: (0, 0)),
            pl.BlockSpec((4 * D, D), lambda i: (0, 0)),
            pl.BlockSpec((1, D), lambda i: (0, 0)),
            pl.BlockSpec((D, 128), lambda i: (0, 0)),
            pl.BlockSpec((1, 128), lambda i: (0, 0)),
        ],
        out_specs=pl.BlockSpec((BN, E), lambda i: (i, 0)),
        out_shape=jax.ShapeDtypeStruct((N, E), jnp.float32),
    )(x, g1, gb1.reshape(1, 4 * D), g2, gb2.reshape(1, D), g3p, gb3p)

    xbf = x.astype(jnp.bfloat16)
    w1bf = W1.astype(jnp.bfloat16)
    w2bf = W2.astype(jnp.bfloat16)
    b1r = b1.reshape(E, 1, H)
    b2r = b2.reshape(E, 1, O)

    y = pl.pallas_call(
        _expert_body,
        grid=(E, N // BN),
        in_specs=[
            pl.BlockSpec((BN, E), lambda e, i: (i, 0)),
            pl.BlockSpec((BN, D), lambda e, i: (i, 0)),
            pl.BlockSpec((1, D, H), lambda e, i: (e, 0, 0)),
            pl.BlockSpec((1, 1, H), lambda e, i: (e, 0, 0)),
            pl.BlockSpec((1, H, O), lambda e, i: (e, 0, 0)),
            pl.BlockSpec((1, 1, O), lambda e, i: (e, 0, 0)),
        ],
        out_specs=pl.BlockSpec((BN, O), lambda e, i: (i, 0)),
        out_shape=jax.ShapeDtypeStruct((N, O), jnp.float32),
        scratch_shapes=[pltpu.VMEM((N, O), jnp.float32)],
    )(p, xbf, w1bf, b1r, w2bf, b2r)

    return (y, p)
